# baseline (speedup 1.0000x reference)
import jax
import jax.numpy as jnp
from jax import lax
from jax.experimental import pallas as pl
from jax.experimental.pallas import tpu as pltpu
from jax.experimental.pallas import tpu_sc as plsc

NUM_CLASSES = 1000
BATCH = 16384

_NC = 2
_NS = 16
_NW = _NC * _NS
_RPW = BATCH // _NW
_C = 32
_NCHUNK = _RPW // _C
_TAIL_SEG = 62


def _sc_body(x_hbm, out_hbm, buf0, buf1, idx_v, sem0, sem1):
    wid = lax.axis_index("s") * _NC + lax.axis_index("c")
    base = pl.multiple_of(wid * _RPW, _RPW)

    pltpu.sync_copy(x_hbm.at[pl.ds(base, _RPW)], idx_v.at[pl.ds(0, _RPW)])

    bufs = (buf0, buf1)
    sems = (sem0, sem1)
    iota16 = lax.iota(jnp.int32, 16)

    def fill_chunk(k, buf):
        def row_body(r, c):
            xv = idx_v[pl.ds(k * _C + r, 16)]
            xr = xv[0]
            tail = pl.multiple_of((xr >> 10) + (_TAIL_SEG * 16), 16)
            for s in range(_TAIL_SEG):
                buf[r, pl.ds(s * 16, 16)] = jnp.where(
                    iota16 + s * 16 == xr, 1.0, 0.0
                )
            buf[r, pl.ds(tail, 16)] = jnp.where(
                iota16 + _TAIL_SEG * 16 == xr, 1.0, 0.0
            )
            return c

        lax.fori_loop(0, _C, row_body, 0)

    def dma(k, buf, sem):
        dst = out_hbm.at[pl.ds(base + k * _C, _C), :]
        return pltpu.async_copy(buf, dst, sem)

    copies = [None, None]
    for k in range(_NCHUNK):
        b = k % 2
        if copies[b] is not None:
            copies[b].wait()
        fill_chunk(k, bufs[b])
        copies[b] = dma(k, bufs[b], sems[b])
    copies[0].wait()
    copies[1].wait()


def kernel(x, W, b):
    xi = x.astype(jnp.int32)
    mesh = plsc.VectorSubcoreMesh(core_axis_name="c", subcore_axis_name="s")
    out_ref = jax.new_ref(jnp.zeros((BATCH, NUM_CLASSES), jnp.float32))
    pl.kernel(
        _sc_body,
        out_type=(),
        mesh=mesh,
        compiler_params=pltpu.CompilerParams(
            use_tc_tiling_on_sc=True, disable_bounds_checks=True
        ),
        scratch_types=[
            pltpu.VMEM((_C, NUM_CLASSES), jnp.float32),
            pltpu.VMEM((_C, NUM_CLASSES), jnp.float32),
            pltpu.VMEM((_RPW + 16,), jnp.int32),
            pltpu.SemaphoreType.DMA,
            pltpu.SemaphoreType.DMA,
        ],
    )(xi, out_ref)
    return out_ref[...]


# speedup vs baseline: 1.1586x; 1.1586x over previous
import jax
import jax.numpy as jnp
from jax import lax
from jax.experimental import pallas as pl
from jax.experimental.pallas import tpu as pltpu
from jax.experimental.pallas import tpu_sc as plsc

NUM_CLASSES = 1000
BATCH = 16384

_NC = 2
_NS = 16
_NW = _NC * _NS
_RPW = BATCH // _NW
_C = 32
_NCHUNK = _RPW // _C
_TAIL_SEG = 62


def _sc_body(x_hbm, out_hbm, buf0, buf1, idx_v, sem0, sem1):
    wid = lax.axis_index("s") * _NC + lax.axis_index("c")
    base = pl.multiple_of(wid * _RPW, _RPW)

    pltpu.sync_copy(x_hbm.at[pl.ds(base, _RPW)], idx_v.at[pl.ds(0, _RPW)])

    bufs = (buf0, buf1)
    sems = (sem0, sem1)
    iota16 = lax.iota(jnp.int32, 16)

    def fill_chunk(k, buf):
        def row_body(r, c):
            xv = idx_v[pl.ds(k * _C + r, 16)]
            xr = xv[0]
            tail = pl.multiple_of((xr >> 10) + (_TAIL_SEG * 16), 16)
            for s in range(_TAIL_SEG):
                buf[r, pl.ds(s * 16, 16)] = jnp.where(
                    iota16 + s * 16 == xr, 1.0, 0.0
                )
            buf[r, pl.ds(tail, 16)] = jnp.where(
                iota16 + _TAIL_SEG * 16 == xr, 1.0, 0.0
            )
            return c

        lax.fori_loop(0, _C, row_body, 0)

    def dma(k, buf, sem):
        dst = out_hbm.at[pl.ds(base + k * _C, _C), :]
        return pltpu.async_copy(buf, dst, sem)

    copies = [None, None]
    for k in range(_NCHUNK):
        b = k % 2
        if copies[b] is not None:
            copies[b].wait()
        fill_chunk(k, bufs[b])
        copies[b] = dma(k, bufs[b], sems[b])
    copies[0].wait()
    copies[1].wait()


def kernel(x, W, b):
    xi = x.astype(jnp.int32)
    mesh = plsc.VectorSubcoreMesh(core_axis_name="c", subcore_axis_name="s")
    out = pl.kernel(
        _sc_body,
        out_type=jax.ShapeDtypeStruct((BATCH, NUM_CLASSES), jnp.float32),
        mesh=mesh,
        compiler_params=pltpu.CompilerParams(
            use_tc_tiling_on_sc=True, disable_bounds_checks=True
        ),
        scratch_types=[
            pltpu.VMEM((_C, NUM_CLASSES), jnp.float32),
            pltpu.VMEM((_C, NUM_CLASSES), jnp.float32),
            pltpu.VMEM((_RPW + 16,), jnp.int32),
            pltpu.SemaphoreType.DMA,
            pltpu.SemaphoreType.DMA,
        ],
    )(xi)
    return out


# speedup vs baseline: 1.1879x; 1.0253x over previous
import jax
import jax.numpy as jnp
from jax import lax
from jax.experimental import pallas as pl
from jax.experimental.pallas import tpu as pltpu
from jax.experimental.pallas import tpu_sc as plsc

NUM_CLASSES = 1000
BATCH = 16384

_NC = 2
_NS = 16
_NW = _NC * _NS
_RPW = BATCH // _NW
_C = 32
_NCHUNK = _RPW // _C
_TAIL_SEG = 62


def _sc_body(x_hbm, out_hbm, buf0, buf1, buf2, idx_v, sem0, sem1, sem2):
    wid = lax.axis_index("s") * _NC + lax.axis_index("c")
    base = pl.multiple_of(wid * _RPW, _RPW)

    pltpu.sync_copy(x_hbm.at[pl.ds(base, _RPW)], idx_v.at[pl.ds(0, _RPW)])

    bufs = (buf0, buf1, buf2)
    sems = (sem0, sem1, sem2)
    iota16 = lax.iota(jnp.int32, 16)

    def fill_chunk(k, buf):
        def row_body(r, c):
            xv = idx_v[pl.ds(k * _C + r, 16)]
            xr = xv[0]
            tail = pl.multiple_of((xr >> 10) + (_TAIL_SEG * 16), 16)
            for s in range(_TAIL_SEG):
                buf[r, pl.ds(s * 16, 16)] = jnp.where(
                    iota16 + s * 16 == xr, 1.0, 0.0
                )
            buf[r, pl.ds(tail, 16)] = jnp.where(
                iota16 + _TAIL_SEG * 16 == xr, 1.0, 0.0
            )
            return c

        lax.fori_loop(0, _C, row_body, 0)

    def dma(k, buf, sem):
        dst = out_hbm.at[pl.ds(base + k * _C, _C), :]
        return pltpu.async_copy(buf, dst, sem)

    copies = [None, None, None]
    for k in range(_NCHUNK):
        b = k % 3
        if copies[b] is not None:
            copies[b].wait()
        fill_chunk(k, bufs[b])
        copies[b] = dma(k, bufs[b], sems[b])
    for cp in copies:
        if cp is not None:
            cp.wait()


def kernel(x, W, b):
    xi = x.astype(jnp.int32)
    mesh = plsc.VectorSubcoreMesh(core_axis_name="c", subcore_axis_name="s")
    out = pl.kernel(
        _sc_body,
        out_type=jax.ShapeDtypeStruct((BATCH, NUM_CLASSES), jnp.float32),
        mesh=mesh,
        compiler_params=pltpu.CompilerParams(
            use_tc_tiling_on_sc=True, disable_bounds_checks=True
        ),
        scratch_types=[
            pltpu.VMEM((_C, NUM_CLASSES), jnp.float32),
            pltpu.VMEM((_C, NUM_CLASSES), jnp.float32),
            pltpu.VMEM((_C, NUM_CLASSES), jnp.float32),
            pltpu.VMEM((_RPW + 16,), jnp.int32),
            pltpu.SemaphoreType.DMA,
            pltpu.SemaphoreType.DMA,
            pltpu.SemaphoreType.DMA,
        ],
    )(xi)
    return out
